# baseline (speedup 1.0000x reference)
import functools

import jax
import jax.numpy as jnp
from jax import lax
from jax.experimental import pallas as pl
from jax.experimental.pallas import tpu as pltpu
from jax.experimental.pallas import tpu_sc as plsc

BATCH = 16384
EMB = 32
HALF = EMB // 2
NC = 2
NS = 16
NW = NC * NS
BPW = BATCH // NW
CHUNK = 128
NCHUNK = BPW // CHUNK
LANES = 16

S_U, SH_U = 131072, 17
S_V, SH_V = 16384, 14
BS = 16384

_MESH = plsc.VectorSubcoreMesh(core_axis_name="c", subcore_axis_name="s")


def _transposer_body(x0, x1, x2, x3, x4, x5, x6, x7, o):
    xa = jnp.concatenate(
        [x0[...], x1[...], x2[...], x3[...]], axis=0).astype(jnp.bfloat16)
    xb = jnp.concatenate(
        [x4[...], x5[...], x6[...], x7[...]], axis=0).astype(jnp.bfloat16)
    r = lax.broadcasted_iota(jnp.int32, (128, 128), 0)
    c = lax.broadcasted_iota(jnp.int32, (128, 128), 1)
    eye = (r == c).astype(jnp.bfloat16)
    dn = (((0,), (0,)), ((), ()))
    ya = lax.dot_general(xa, eye, dn, preferred_element_type=jnp.float32)
    yb = lax.dot_general(xb, eye, dn, preferred_element_type=jnp.float32)
    lo = lax.bitcast_convert_type(
        ya.astype(jnp.bfloat16), jnp.uint16).astype(jnp.uint32)
    hi = lax.bitcast_convert_type(
        yb.astype(jnp.bfloat16), jnp.uint16).astype(jnp.uint32)
    o[...] = lax.bitcast_convert_type(lo | (hi << 16), jnp.float32)


def _widen(table, s):
    n = table.shape[0]
    tab_t = table.T
    k = s // BS
    last = (n + BS - 1) // BS - 1
    specs = [
        pl.BlockSpec(
            (EMB, BS),
            functools.partial(lambda u, i: (0, jnp.minimum(u * k + i, last)), u))
        for u in range(8)
    ]
    return pl.pallas_call(
        _transposer_body,
        grid=(k,),
        in_specs=specs,
        out_specs=pl.BlockSpec((BS, 128), lambda i: (i, 0)),
        out_shape=jax.ShapeDtypeStruct((s, 128), jnp.float32),
    )(*([tab_t] * 8))


def _gather_extract(tab_hbm, ids_v, out_hbm, shift, mask, base,
                    widx_v, rows_v, ext_v, gsem):
    for j in range(NCHUNK):
        def wbody(g, _, j=j):
            v = ids_v[j, pl.ds(g * LANES, LANES)]
            widx_v[j, pl.ds(g * LANES, LANES)] = v & mask
            return 0
        lax.fori_loop(0, CHUNK // LANES, wbody, 0)
    copies = [
        pltpu.async_copy(tab_hbm.at[widx_v.at[j]],
                         rows_v.at[pl.ds(j * CHUNK, CHUNK)], gsem)
        for j in range(NCHUNK)
    ]
    lane = lax.iota(jnp.int32, LANES)
    himask = jnp.int32(-65536)
    for j in range(NCHUNK):
        copies[j].wait()
        def ebody(g, _, j=j):
            ids16 = ids_v[j, pl.ds(g * LANES, LANES)]
            r = j * CHUNK + g * LANES + lane
            u = ids16 >> shift
            cbase = (u & 3) << 5
            take_hi = u >= 4
            for m in range(EMB):
                vals = plsc.load_gather(rows_v, [r, cbase + m])
                vi = plsc.bitcast(vals, jnp.int32)
                lo = plsc.bitcast(vi << 16, jnp.float32)
                hi = plsc.bitcast(vi & himask, jnp.float32)
                plsc.store_scatter(ext_v, [lane * 0 + m, r],
                                   jnp.where(take_hi, hi, lo))
            return 0
        lax.fori_loop(0, CHUNK // LANES, ebody, 0)
    pltpu.sync_copy(ext_v, out_hbm.at[:, pl.ds(base, BPW)])


def _make_sc_gather(shift, mask):
    @functools.partial(
        pl.kernel,
        out_type=jax.ShapeDtypeStruct((EMB, BATCH), jnp.float32),
        mesh=_MESH,
        compiler_params=pltpu.CompilerParams(needs_layout_passes=False),
        scratch_types=[
            pltpu.VMEM((NCHUNK, CHUNK), jnp.int32),
            pltpu.VMEM((NCHUNK, CHUNK), jnp.int32),
            pltpu.VMEM((BPW, 128), jnp.float32),
            pltpu.VMEM((EMB, BPW), jnp.float32),
            pltpu.SemaphoreType.DMA,
        ],
    )
    def sc_gather(ids_hbm, tab_hbm, out_hbm, idx_v, widx_v, rows_v, ext_v, gsem):
        wid = lax.axis_index("s") * NC + lax.axis_index("c")
        base = wid * BPW
        pltpu.sync_copy(ids_hbm.at[wid], idx_v)
        _gather_extract(tab_hbm, idx_v, out_hbm, shift, mask, base,
                        widx_v, rows_v, ext_v, gsem)
    return sc_gather


_sc_gather_u = _make_sc_gather(SH_U, S_U - 1)
_sc_gather_v = _make_sc_gather(SH_V, S_V - 1)


_ROWS = 16384


def _mlp_body(uet, vet, w1a, w1b, b1, wo1, bo1, wo2, bo2, l1, l2):
    dn0 = (((0,), (0,)), ((), ()))
    h = lax.dot_general(uet[...], w1a[...], dn0,
                        preferred_element_type=jnp.float32)
    h += lax.dot_general(vet[...], w1b[...], dn0,
                         preferred_element_type=jnp.float32)
    h = jnp.maximum(h + b1[...], 0.0)
    dn1 = (((0,), (1,)), ((), ()))
    l1[...] = lax.dot_general(wo1[...], h, dn1,
                              preferred_element_type=jnp.float32) + bo1[...]
    l2[...] = lax.dot_general(wo2[...], h, dn1,
                              preferred_element_type=jnp.float32) + bo2[...]


def _mlp(uet, vet, w1a, w1b, b1, wo1, bo1, wo2, bo2):
    grid = (BATCH // _ROWS,)
    full = lambda shape: pl.BlockSpec(shape, lambda i: (0, 0))
    return pl.pallas_call(
        _mlp_body,
        grid=grid,
        in_specs=[
            pl.BlockSpec((EMB, _ROWS), lambda i: (0, i)),
            pl.BlockSpec((EMB, _ROWS), lambda i: (0, i)),
            full((EMB, 32)),
            full((EMB, 32)),
            full((1, 32)),
            full((32, 10)),
            full((10, 1)),
            full((32, 1)),
            full((1, 1)),
        ],
        out_specs=[
            pl.BlockSpec((10, _ROWS), lambda i: (0, i)),
            pl.BlockSpec((1, _ROWS), lambda i: (0, i)),
        ],
        out_shape=[
            jax.ShapeDtypeStruct((10, BATCH), jnp.float32),
            jax.ShapeDtypeStruct((1, BATCH), jnp.float32),
        ],
    )(uet, vet, w1a, w1b, b1, wo1, bo1, wo2, bo2)


def kernel(user_id, video_id, user_table, video_table, W1, b1, Wo1, bo1, Wo2, bo2):
    uid = jnp.asarray(user_id, jnp.int32)
    vid = jnp.asarray(video_id, jnp.int32)
    vwide = _widen(video_table, S_V)
    vet = _sc_gather_v(vid.reshape(NW, NCHUNK, CHUNK), vwide)
    ut, _ = lax.optimization_barrier((user_table, vwide))
    uwide = _widen(ut, S_U)
    uet = _sc_gather_u(uid.reshape(NW, NCHUNK, CHUNK), uwide)
    l1t, l2t = _mlp(uet, vet, W1[:EMB], W1[EMB:], b1.reshape(1, 32),
                    Wo1, bo1.reshape(10, 1), Wo2, bo2.reshape(1, 1))
    return (l1t.T, l2t.T)


# speedup vs baseline: 1.0217x; 1.0217x over previous
import functools

import jax
import jax.numpy as jnp
from jax import lax
from jax.experimental import pallas as pl
from jax.experimental.pallas import tpu as pltpu
from jax.experimental.pallas import tpu_sc as plsc

BATCH = 16384
EMB = 32
HALF = EMB // 2
NC = 2
NS = 16
NW = NC * NS
BPW = BATCH // NW
CHUNK = 128
NCHUNK = BPW // CHUNK
LANES = 16

S_U, SH_U = 131072, 17
S_V, SH_V = 16384, 14
BS = 8192

_MESH = plsc.VectorSubcoreMesh(core_axis_name="c", subcore_axis_name="s")


def _transposer_body(x0, x1, x2, x3, x4, x5, x6, x7, o):
    xa = jnp.concatenate(
        [x0[...], x1[...], x2[...], x3[...]], axis=0).astype(jnp.bfloat16)
    xb = jnp.concatenate(
        [x4[...], x5[...], x6[...], x7[...]], axis=0).astype(jnp.bfloat16)
    r = lax.broadcasted_iota(jnp.int32, (128, 128), 0)
    c = lax.broadcasted_iota(jnp.int32, (128, 128), 1)
    eye = (r == c).astype(jnp.bfloat16)
    dn = (((0,), (0,)), ((), ()))
    ya = lax.dot_general(xa, eye, dn, preferred_element_type=jnp.float32)
    yb = lax.dot_general(xb, eye, dn, preferred_element_type=jnp.float32)
    lo = lax.bitcast_convert_type(
        ya.astype(jnp.bfloat16), jnp.uint16).astype(jnp.uint32)
    hi = lax.bitcast_convert_type(
        yb.astype(jnp.bfloat16), jnp.uint16).astype(jnp.uint32)
    o[...] = lax.bitcast_convert_type(lo | (hi << 16), jnp.float32)


def _widen(table, s):
    n = table.shape[0]
    tab_t = table.T
    k = s // BS
    last = (n + BS - 1) // BS - 1
    specs = [
        pl.BlockSpec(
            (EMB, BS),
            functools.partial(lambda u, i: (0, jnp.minimum(u * k + i, last)), u))
        for u in range(8)
    ]
    return pl.pallas_call(
        _transposer_body,
        grid=(k,),
        in_specs=specs,
        out_specs=pl.BlockSpec((BS, 128), lambda i: (i, 0)),
        out_shape=jax.ShapeDtypeStruct((s, 128), jnp.float32),
    )(*([tab_t] * 8))


def _gather_extract(tab_hbm, ids_v, out_hbm, shift, mask, base,
                    widx_v, rows_v, ext_v, gsem):
    for j in range(NCHUNK):
        def wbody(g, _, j=j):
            v = ids_v[j, pl.ds(g * LANES, LANES)]
            widx_v[j, pl.ds(g * LANES, LANES)] = v & mask
            return 0
        lax.fori_loop(0, CHUNK // LANES, wbody, 0)
    copies = [
        pltpu.async_copy(tab_hbm.at[widx_v.at[j]],
                         rows_v.at[pl.ds(j * CHUNK, CHUNK)], gsem)
        for j in range(NCHUNK)
    ]
    lane = lax.iota(jnp.int32, LANES)
    himask = jnp.int32(-65536)
    for j in range(NCHUNK):
        copies[j].wait()
        def ebody(g2, _, j=j):
            for h in range(2):
                g = g2 * 2 + h
                ids16 = ids_v[j, pl.ds(g * LANES, LANES)]
                r = j * CHUNK + g * LANES + lane
                u = ids16 >> shift
                cbase = (u & 3) << 5
                take_hi = u >= 4
                for m in range(EMB):
                    vals = plsc.load_gather(rows_v, [r, cbase + m])
                    vi = plsc.bitcast(vals, jnp.int32)
                    lo = plsc.bitcast(vi << 16, jnp.float32)
                    hi = plsc.bitcast(vi & himask, jnp.float32)
                    plsc.store_scatter(ext_v, [lane * 0 + m, r],
                                       jnp.where(take_hi, hi, lo))
            return 0
        lax.fori_loop(0, CHUNK // LANES // 2, ebody, 0)
    pltpu.sync_copy(ext_v, out_hbm.at[:, pl.ds(base, BPW)])


def _make_sc_gather(shift, mask):
    @functools.partial(
        pl.kernel,
        out_type=jax.ShapeDtypeStruct((EMB, BATCH), jnp.float32),
        mesh=_MESH,
        compiler_params=pltpu.CompilerParams(needs_layout_passes=False),
        scratch_types=[
            pltpu.VMEM((NCHUNK, CHUNK), jnp.int32),
            pltpu.VMEM((NCHUNK, CHUNK), jnp.int32),
            pltpu.VMEM((BPW, 128), jnp.float32),
            pltpu.VMEM((EMB, BPW), jnp.float32),
            pltpu.SemaphoreType.DMA,
        ],
    )
    def sc_gather(ids_hbm, tab_hbm, out_hbm, idx_v, widx_v, rows_v, ext_v, gsem):
        wid = lax.axis_index("s") * NC + lax.axis_index("c")
        base = wid * BPW
        pltpu.sync_copy(ids_hbm.at[wid], idx_v)
        _gather_extract(tab_hbm, idx_v, out_hbm, shift, mask, base,
                        widx_v, rows_v, ext_v, gsem)
    return sc_gather


_sc_gather_u = _make_sc_gather(SH_U, S_U - 1)
_sc_gather_v = _make_sc_gather(SH_V, S_V - 1)


_ROWS = 16384


def _mlp_body(uet, vet, w1a, w1b, b1, wo1, bo1, wo2, bo2, l1, l2):
    dn0 = (((0,), (0,)), ((), ()))
    h = lax.dot_general(uet[...], w1a[...], dn0,
                        preferred_element_type=jnp.float32)
    h += lax.dot_general(vet[...], w1b[...], dn0,
                         preferred_element_type=jnp.float32)
    h = jnp.maximum(h + b1[...], 0.0)
    dn1 = (((0,), (1,)), ((), ()))
    l1[...] = lax.dot_general(wo1[...], h, dn1,
                              preferred_element_type=jnp.float32) + bo1[...]
    l2[...] = lax.dot_general(wo2[...], h, dn1,
                              preferred_element_type=jnp.float32) + bo2[...]


def _mlp(uet, vet, w1a, w1b, b1, wo1, bo1, wo2, bo2):
    grid = (BATCH // _ROWS,)
    full = lambda shape: pl.BlockSpec(shape, lambda i: (0, 0))
    return pl.pallas_call(
        _mlp_body,
        grid=grid,
        in_specs=[
            pl.BlockSpec((EMB, _ROWS), lambda i: (0, i)),
            pl.BlockSpec((EMB, _ROWS), lambda i: (0, i)),
            full((EMB, 32)),
            full((EMB, 32)),
            full((1, 32)),
            full((32, 10)),
            full((10, 1)),
            full((32, 1)),
            full((1, 1)),
        ],
        out_specs=[
            pl.BlockSpec((10, _ROWS), lambda i: (0, i)),
            pl.BlockSpec((1, _ROWS), lambda i: (0, i)),
        ],
        out_shape=[
            jax.ShapeDtypeStruct((10, BATCH), jnp.float32),
            jax.ShapeDtypeStruct((1, BATCH), jnp.float32),
        ],
    )(uet, vet, w1a, w1b, b1, wo1, bo1, wo2, bo2)


def kernel(user_id, video_id, user_table, video_table, W1, b1, Wo1, bo1, Wo2, bo2):
    uid = jnp.asarray(user_id, jnp.int32)
    vid = jnp.asarray(video_id, jnp.int32)
    vwide = _widen(video_table, S_V)
    vet = _sc_gather_v(vid.reshape(NW, NCHUNK, CHUNK), vwide)
    ut, _ = lax.optimization_barrier((user_table, vwide))
    uwide = _widen(ut, S_U)
    uet = _sc_gather_u(uid.reshape(NW, NCHUNK, CHUNK), uwide)
    l1t, l2t = _mlp(uet, vet, W1[:EMB], W1[EMB:], b1.reshape(1, 32),
                    Wo1, bo1.reshape(10, 1), Wo2, bo2.reshape(1, 1))
    return (l1t.T, l2t.T)


# speedup vs baseline: 1.0238x; 1.0021x over previous
import functools

import jax
import jax.numpy as jnp
from jax import lax
from jax.experimental import pallas as pl
from jax.experimental.pallas import tpu as pltpu
from jax.experimental.pallas import tpu_sc as plsc

BATCH = 16384
EMB = 32
HALF = EMB // 2
NC = 2
NS = 16
NW = NC * NS
BPW = BATCH // NW
CHUNK = 128
NCHUNK = BPW // CHUNK
LANES = 16

S_U, SH_U = 131072, 17
S_V, SH_V = 16384, 14
BS = 8192

_MESH = plsc.VectorSubcoreMesh(core_axis_name="c", subcore_axis_name="s")


def _transposer_body(x0, x1, x2, x3, x4, x5, x6, x7, o):
    xa = jnp.concatenate(
        [x0[...], x1[...], x2[...], x3[...]], axis=0).astype(jnp.bfloat16)
    xb = jnp.concatenate(
        [x4[...], x5[...], x6[...], x7[...]], axis=0).astype(jnp.bfloat16)
    r = lax.broadcasted_iota(jnp.int32, (128, 128), 0)
    c = lax.broadcasted_iota(jnp.int32, (128, 128), 1)
    eye = (r == c).astype(jnp.bfloat16)
    dn = (((0,), (0,)), ((), ()))
    ya = lax.dot_general(xa, eye, dn, preferred_element_type=jnp.float32)
    yb = lax.dot_general(xb, eye, dn, preferred_element_type=jnp.float32)
    lo = lax.bitcast_convert_type(
        ya.astype(jnp.bfloat16), jnp.uint16).astype(jnp.uint32)
    hi = lax.bitcast_convert_type(
        yb.astype(jnp.bfloat16), jnp.uint16).astype(jnp.uint32)
    o[...] = lax.bitcast_convert_type(lo | (hi << 16), jnp.float32)


def _widen(table, s):
    n = table.shape[0]
    tab_t = table.T
    k = s // BS
    last = (n + BS - 1) // BS - 1
    specs = [
        pl.BlockSpec(
            (EMB, BS),
            functools.partial(lambda u, i: (0, jnp.minimum(u * k + i, last)), u))
        for u in range(8)
    ]
    return pl.pallas_call(
        _transposer_body,
        grid=(k,),
        in_specs=specs,
        out_specs=pl.BlockSpec((BS, 128), lambda i: (i, 0)),
        out_shape=jax.ShapeDtypeStruct((s, 128), jnp.float32),
    )(*([tab_t] * 8))


def _gather_extract(tab_hbm, ids_v, out_hbm, shift, mask, base,
                    widx_v, rows_v, ext_v, gsem):
    for j in range(NCHUNK):
        def wbody(g, _, j=j):
            v = ids_v[j, pl.ds(g * LANES, LANES)]
            widx_v[j, pl.ds(g * LANES, LANES)] = v & mask
            return 0
        lax.fori_loop(0, CHUNK // LANES, wbody, 0)
    copies = [
        pltpu.async_copy(tab_hbm.at[widx_v.at[j]],
                         rows_v.at[pl.ds(j * CHUNK, CHUNK)], gsem)
        for j in range(NCHUNK)
    ]
    lane = lax.iota(jnp.int32, LANES)
    himask = jnp.int32(-65536)
    for j in range(NCHUNK):
        copies[j].wait()
        def ebody(g2, _, j=j):
            for h in range(2):
                g = g2 * 2 + h
                ids16 = ids_v[j, pl.ds(g * LANES, LANES)]
                r = j * CHUNK + g * LANES + lane
                u = ids16 >> shift
                cbase = (u & 3) << 5
                take_hi = u >= 4
                for m in range(EMB):
                    vals = plsc.load_gather(rows_v, [r, cbase + m])
                    vi = plsc.bitcast(vals, jnp.int32)
                    lo = plsc.bitcast(vi << 16, jnp.float32)
                    hi = plsc.bitcast(vi & himask, jnp.float32)
                    plsc.store_scatter(ext_v, [lane * 0 + m, r],
                                       jnp.where(take_hi, hi, lo))
            return 0
        lax.fori_loop(0, CHUNK // LANES // 2, ebody, 0)
    pltpu.sync_copy(ext_v, out_hbm.at[:, pl.ds(base, BPW)])


def _make_sc_gather(shift, mask):
    @functools.partial(
        pl.kernel,
        out_type=jax.ShapeDtypeStruct((EMB, BATCH), jnp.float32),
        mesh=_MESH,
        compiler_params=pltpu.CompilerParams(needs_layout_passes=False),
        scratch_types=[
            pltpu.VMEM((NCHUNK, CHUNK), jnp.int32),
            pltpu.VMEM((NCHUNK, CHUNK), jnp.int32),
            pltpu.VMEM((BPW, 128), jnp.float32),
            pltpu.VMEM((EMB, BPW), jnp.float32),
            pltpu.SemaphoreType.DMA,
        ],
    )
    def sc_gather(ids_hbm, tab_hbm, out_hbm, idx_v, widx_v, rows_v, ext_v, gsem):
        wid = lax.axis_index("s") * NC + lax.axis_index("c")
        base = wid * BPW
        pltpu.sync_copy(ids_hbm.at[wid], idx_v)
        _gather_extract(tab_hbm, idx_v, out_hbm, shift, mask, base,
                        widx_v, rows_v, ext_v, gsem)
    return sc_gather


_sc_gather_u = _make_sc_gather(SH_U, S_U - 1)
_sc_gather_v = _make_sc_gather(SH_V, S_V - 1)


_ROWS = 8192


def _mlp_body(uet, vet, w1a, w1b, b1, wo1, bo1, wo2, bo2, l1, l2):
    dn0 = (((0,), (0,)), ((), ()))
    h = lax.dot_general(uet[...], w1a[...], dn0,
                        preferred_element_type=jnp.float32)
    h += lax.dot_general(vet[...], w1b[...], dn0,
                         preferred_element_type=jnp.float32)
    h = jnp.maximum(h + b1[...], 0.0)
    dn1 = (((0,), (1,)), ((), ()))
    l1[...] = lax.dot_general(wo1[...], h, dn1,
                              preferred_element_type=jnp.float32) + bo1[...]
    l2[...] = lax.dot_general(wo2[...], h, dn1,
                              preferred_element_type=jnp.float32) + bo2[...]


def _mlp(uet, vet, w1a, w1b, b1, wo1, bo1, wo2, bo2):
    grid = (BATCH // _ROWS,)
    full = lambda shape: pl.BlockSpec(shape, lambda i: (0, 0))
    return pl.pallas_call(
        _mlp_body,
        grid=grid,
        in_specs=[
            pl.BlockSpec((EMB, _ROWS), lambda i: (0, i)),
            pl.BlockSpec((EMB, _ROWS), lambda i: (0, i)),
            full((EMB, 32)),
            full((EMB, 32)),
            full((1, 32)),
            full((32, 10)),
            full((10, 1)),
            full((32, 1)),
            full((1, 1)),
        ],
        out_specs=[
            pl.BlockSpec((10, _ROWS), lambda i: (0, i)),
            pl.BlockSpec((1, _ROWS), lambda i: (0, i)),
        ],
        out_shape=[
            jax.ShapeDtypeStruct((10, BATCH), jnp.float32),
            jax.ShapeDtypeStruct((1, BATCH), jnp.float32),
        ],
    )(uet, vet, w1a, w1b, b1, wo1, bo1, wo2, bo2)


def kernel(user_id, video_id, user_table, video_table, W1, b1, Wo1, bo1, Wo2, bo2):
    uid = jnp.asarray(user_id, jnp.int32)
    vid = jnp.asarray(video_id, jnp.int32)
    vwide = _widen(video_table, S_V)
    vet = _sc_gather_v(vid.reshape(NW, NCHUNK, CHUNK), vwide)
    ut, _ = lax.optimization_barrier((user_table, vwide))
    uwide = _widen(ut, S_U)
    uet = _sc_gather_u(uid.reshape(NW, NCHUNK, CHUNK), uwide)
    l1t, l2t = _mlp(uet, vet, W1[:EMB], W1[EMB:], b1.reshape(1, 32),
                    Wo1, bo1.reshape(10, 1), Wo2, bo2.reshape(1, 1))
    return (l1t.T, l2t.T)
